# baseline (device time: 412812 ns/iter reference)
import jax
import jax.numpy as jnp
from jax import lax
from jax.experimental import pallas as pl
from jax.experimental.pallas import tpu as pltpu

_TILES = 8


def kernel(x):
    m, n = x.shape
    half = n // 2
    rows = m // _TILES

    def body(x_ref, out_ref, vsend, vrecv, vloc,
             snd_ld_sems, rcv_st_sems, loc_ld_sems, loc_st_sems,
             send_sems, recv_sems, credit_sems):
        my_x = lax.axis_index("x")
        my_y = lax.axis_index("y")
        my_z = lax.axis_index("z")
        peer_y = 1 - my_y
        peer = (my_x, peer_y, my_z)

        barrier = pltpu.get_barrier_semaphore()
        pl.semaphore_signal(
            barrier, inc=1, device_id=peer,
            device_id_type=pl.DeviceIdType.MESH,
        )
        pl.semaphore_wait(barrier, 1)

        rdmas = []
        recv_stores = []
        loc_stores = []
        for i in range(_TILES + 1):
            s = i % 2
            if i < _TILES:
                if i >= 2:
                    rdmas[i - 2].wait_send()
                load_snd = pltpu.make_async_copy(
                    x_ref.at[pl.ds(i * rows, rows), pl.ds(peer_y * half, half)],
                    vsend.at[s],
                    snd_ld_sems.at[s],
                )
                load_snd.start()
                if i >= 2:
                    loc_stores[i - 2].wait()
                load_loc = pltpu.make_async_copy(
                    x_ref.at[pl.ds(i * rows, rows), pl.ds(my_y * half, half)],
                    vloc.at[s],
                    loc_ld_sems.at[s],
                )
                load_loc.start()
                if i >= 2:
                    recv_stores[i - 2].wait()
                    pl.semaphore_signal(
                        credit_sems.at[s], inc=1, device_id=peer,
                        device_id_type=pl.DeviceIdType.MESH,
                    )
                    pl.semaphore_wait(credit_sems.at[s], 1)
                load_snd.wait()
                rdma = pltpu.make_async_remote_copy(
                    src_ref=vsend.at[s],
                    dst_ref=vrecv.at[s],
                    send_sem=send_sems.at[s],
                    recv_sem=recv_sems.at[s],
                    device_id=peer,
                    device_id_type=pl.DeviceIdType.MESH,
                )
                rdma.start()
                rdmas.append(rdma)
                load_loc.wait()
                st_loc = pltpu.make_async_copy(
                    vloc.at[s],
                    out_ref.at[pl.ds(my_y * m + i * rows, rows), :],
                    loc_st_sems.at[s],
                )
                st_loc.start()
                loc_stores.append(st_loc)
            if i >= 1:
                j = i - 1
                rdmas[j].wait_recv()
                st = pltpu.make_async_copy(
                    vrecv.at[j % 2],
                    out_ref.at[pl.ds(peer_y * m + j * rows, rows), :],
                    rcv_st_sems.at[j % 2],
                )
                st.start()
                recv_stores.append(st)

        loc_stores[-2].wait()
        loc_stores[-1].wait()
        recv_stores[-2].wait()
        recv_stores[-1].wait()
        rdmas[-2].wait_send()
        rdmas[-1].wait_send()

    return pl.pallas_call(
        body,
        out_shape=jax.ShapeDtypeStruct((2 * m, half), jnp.float32),
        in_specs=[pl.BlockSpec(memory_space=pl.ANY)],
        out_specs=pl.BlockSpec(memory_space=pl.ANY),
        scratch_shapes=[
            pltpu.VMEM((2, rows, half), jnp.float32),
            pltpu.VMEM((2, rows, half), jnp.float32),
            pltpu.VMEM((2, rows, half), jnp.float32),
            pltpu.SemaphoreType.DMA((2,)),
            pltpu.SemaphoreType.DMA((2,)),
            pltpu.SemaphoreType.DMA((2,)),
            pltpu.SemaphoreType.DMA((2,)),
            pltpu.SemaphoreType.DMA((2,)),
            pltpu.SemaphoreType.DMA((2,)),
            pltpu.SemaphoreType.REGULAR((2,)),
        ],
        compiler_params=pltpu.CompilerParams(collective_id=0),
    )(x)


# device time: 409212 ns/iter; 1.0088x vs baseline; 1.0088x over previous
import jax
import jax.numpy as jnp
from jax import lax
from jax.experimental import pallas as pl
from jax.experimental.pallas import tpu as pltpu

_TILES = 8


def kernel(x):
    m, n = x.shape
    half = n // 2
    rows = m // _TILES

    def body(x_ref, out_ref, vbuf, in_sems, out_sems, send_sem, recv_sem):
        my_x = lax.axis_index("x")
        my_y = lax.axis_index("y")
        my_z = lax.axis_index("z")
        peer_y = 1 - my_y

        barrier = pltpu.get_barrier_semaphore()
        pl.semaphore_signal(
            barrier,
            inc=1,
            device_id=(my_x, peer_y, my_z),
            device_id_type=pl.DeviceIdType.MESH,
        )
        pl.semaphore_wait(barrier, 1)

        rdma = pltpu.make_async_remote_copy(
            src_ref=x_ref.at[:, pl.ds(peer_y * half, half)],
            dst_ref=out_ref.at[pl.ds(my_y * m, m), :],
            send_sem=send_sem,
            recv_sem=recv_sem,
            device_id=(my_x, peer_y, my_z),
            device_id_type=pl.DeviceIdType.MESH,
        )
        rdma.start()

        stores = []
        for i in range(_TILES):
            s = i % 2
            if i >= 2:
                stores[i - 2].wait()
            load = pltpu.make_async_copy(
                x_ref.at[pl.ds(i * rows, rows), pl.ds(my_y * half, half)],
                vbuf.at[s],
                in_sems.at[s],
            )
            load.start()
            load.wait()
            store = pltpu.make_async_copy(
                vbuf.at[s],
                out_ref.at[pl.ds(my_y * m + i * rows, rows), :],
                out_sems.at[s],
            )
            store.start()
            stores.append(store)
        stores[-2].wait()
        stores[-1].wait()

        rdma.wait()

    return pl.pallas_call(
        body,
        out_shape=jax.ShapeDtypeStruct((2 * m, half), jnp.float32),
        in_specs=[pl.BlockSpec(memory_space=pl.ANY)],
        out_specs=pl.BlockSpec(memory_space=pl.ANY),
        scratch_shapes=[
            pltpu.VMEM((2, rows, half), jnp.float32),
            pltpu.SemaphoreType.DMA((2,)),
            pltpu.SemaphoreType.DMA((2,)),
            pltpu.SemaphoreType.DMA,
            pltpu.SemaphoreType.DMA,
        ],
        compiler_params=pltpu.CompilerParams(collective_id=0),
    )(x)
